# baseline (speedup 1.0000x reference)
import functools

import jax
import jax.numpy as jnp
from jax import lax
from jax.experimental import pallas as pl
from jax.experimental.pallas import tpu as pltpu
from jax.experimental.pallas import tpu_sc as plsc

B = 16384
D = 32
NC = 2
NS = 16
L = 16
NW = NC * NS
BPW = B // NW

_mesh = plsc.VectorSubcoreMesh(core_axis_name="c", subcore_axis_name="s")


@functools.partial(
    pl.kernel,
    mesh=_mesh,
    compiler_params=pltpu.CompilerParams(use_tc_tiling_on_sc=False),
    out_type=(
        jax.ShapeDtypeStruct((B,), jnp.float32),
        jax.ShapeDtypeStruct((NW, L), jnp.float32),
    ),
    scratch_types=(
        pltpu.VMEM((BPW,), jnp.int32),
        pltpu.VMEM((BPW,), jnp.int32),
        pltpu.VMEM((BPW, D), jnp.float32),
        pltpu.VMEM((BPW, D), jnp.float32),
        pltpu.VMEM((BPW,), jnp.float32),
        pltpu.VMEM((BPW,), jnp.float32),
        pltpu.VMEM((BPW,), jnp.float32),
        pltpu.VMEM((L,), jnp.float32),
        pltpu.SemaphoreType.DMA,
        pltpu.SemaphoreType.DMA,
        pltpu.SemaphoreType.DMA,
        pltpu.SemaphoreType.DMA,
    ),
)
def _sc_embed(users_hbm, movies_hbm, bu_hbm, bm_hbm, uw_hbm, mw_hbm,
              out_hbm, part_hbm,
              idx_u, idx_m, u_rows, m_rows, bu_v, bm_v, out_v, acc_v,
              s1, s2, s3, s4):
    wid = lax.axis_index("s") * NC + lax.axis_index("c")
    base = wid * BPW

    pltpu.sync_copy(users_hbm.at[pl.ds(base, BPW)], idx_u)
    pltpu.sync_copy(movies_hbm.at[pl.ds(base, BPW)], idx_m)

    c_u = pltpu.async_copy(uw_hbm.at[idx_u], u_rows, s1)
    c_m = pltpu.async_copy(mw_hbm.at[idx_m], m_rows, s2)
    c_bu = pltpu.async_copy(bu_hbm.at[idx_u], bu_v, s3)
    c_bm = pltpu.async_copy(bm_hbm.at[idx_m], bm_v, s4)

    c_bu.wait()
    c_bm.wait()

    def bias_body(k, carry):
        off = pl.multiple_of(k * L, L)
        out_v[pl.ds(off, L)] = bu_v[pl.ds(off, L)] + bm_v[pl.ds(off, L)]
        return carry
    lax.fori_loop(0, BPW // L, bias_body, 0)
    pltpu.sync_copy(out_v, out_hbm.at[pl.ds(base, BPW)])

    c_u.wait()
    c_m.wait()

    def row_body(i, acc):
        u0 = u_rows[i, pl.ds(0, L)]
        m0 = m_rows[i, pl.ds(0, L)]
        u1 = u_rows[i, pl.ds(L, L)]
        m1 = m_rows[i, pl.ds(L, L)]
        d0 = u0 - m0
        d1 = u1 - m1
        q0 = d0 * d0
        q1 = d1 * d1
        return acc + q0 * q0 + q1 * q1
    acc = lax.fori_loop(0, BPW, row_body, jnp.zeros((L,), jnp.float32))
    acc_v[...] = acc
    pltpu.sync_copy(acc_v, part_hbm.at[wid])


def kernel(x, Bu, Bm, u_weight, m_weight):
    users = x[:, 0]
    movies = x[:, 1]
    out, parts = _sc_embed(users, movies, Bu, Bm, u_weight, m_weight)
    return out - jnp.sqrt(jnp.sum(parts))
